# Initial kernel scaffold; baseline (speedup 1.0000x reference)
import jax
import jax.numpy as jnp
from jax.experimental import pallas as pl


def kernel(x, edge_index, edge_weight, batch_ids, speed, W1, b1, W2, b2, Ws, bs, Wl1, bl1, Wl2, bl2):
    raise NotImplementedError("write your pallas kernel here")


# speedup vs baseline: 7.4456x; 7.4456x over previous
import functools

import jax
import jax.numpy as jnp
from jax import lax
from jax.experimental import pallas as pl
from jax.experimental.pallas import tpu as pltpu
from jax.experimental.pallas import tpu_sc as plsc

N_NODES = 10000
N_EDGES = 320000
N_GRAPHS = 16
D = 128

NC = 2
NS = 16
NW = NC * NS
K = 128
C = 80
CB = 16
E_PAD = NW * C * K
N_PAD = 10240
STRIPE = N_PAD // NS

_mesh = lambda: plsc.VectorSubcoreMesh(
    core_axis_name="c", subcore_axis_name="s", num_cores=NC, num_subcores=NS)


def _sc_deg(dst3, w3):
  @functools.partial(
      pl.kernel,
      out_type=jax.ShapeDtypeStruct((NC, N_PAD), jnp.float32),
      mesh=_mesh(),
      scratch_types=[
          pltpu.VMEM((C, K), jnp.int32),
          pltpu.VMEM((C, K), jnp.float32),
          pltpu.VMEM((STRIPE,), jnp.float32),
          pltpu.VMEM_SHARED((N_PAD,), jnp.float32),
      ],
  )
  def k(dst_hbm, w_hbm, out_hbm, idx_v, w_v, zero_v, acc_sh):
    c = lax.axis_index("c")
    s = lax.axis_index("s")
    w = s * NC + c
    z16 = jnp.zeros((16,), jnp.float32)

    def zinit(i, _):
      zero_v[pl.ds(i * 16, 16)] = z16
      return 0

    lax.fori_loop(0, STRIPE // 16, zinit, 0)
    pltpu.sync_copy(zero_v, acc_sh.at[pl.ds(s * STRIPE, STRIPE)])
    plsc.subcore_barrier()

    pltpu.sync_copy(dst_hbm.at[w], idx_v)
    pltpu.sync_copy(w_hbm.at[w], w_v)

    def body(i, _):
      pltpu.sync_copy(w_v.at[i], acc_sh.at[idx_v.at[i]], add=True)
      return 0

    lax.fori_loop(0, C, body, 0)
    plsc.subcore_barrier()
    pltpu.sync_copy(acc_sh.at[pl.ds(s * STRIPE, STRIPE)],
                    out_hbm.at[c, pl.ds(s * STRIPE, STRIPE)])

  return k(dst3, w3)


def _sc_spmm(xw, src3, dst3, norm3):
  @functools.partial(
      pl.kernel,
      out_type=jax.ShapeDtypeStruct((NC, N_PAD, D), jnp.float32),
      mesh=_mesh(),
      scratch_types=[
          pltpu.VMEM((CB, K), jnp.int32),
          pltpu.VMEM((CB, K), jnp.int32),
          pltpu.VMEM((CB, K), jnp.float32),
          pltpu.VMEM((K, D), jnp.float32),
          pltpu.VMEM_SHARED((N_PAD, D), jnp.float32),
          pltpu.SemaphoreType.DMA,
      ],
  )
  def k(xw_hbm, src_hbm, dst_hbm, norm_hbm, out_hbm,
        sidx_v, didx_v, norm_v, rows_v, acc_sh, sem):
    c = lax.axis_index("c")
    s = lax.axis_index("s")
    w = s * NC + c
    z16 = jnp.zeros((16,), jnp.float32)

    def zinit(i, _):
      for j in range(D // 16):
        rows_v[i, pl.ds(j * 16, 16)] = z16
      return 0

    lax.fori_loop(0, K, zinit, 0)
    for t in range(STRIPE // K):
      pltpu.sync_copy(rows_v, acc_sh.at[pl.ds(s * STRIPE + t * K, K)])

    def group(g, _):
      base = g * CB
      pltpu.sync_copy(src_hbm.at[w, pl.ds(base, CB)], sidx_v)
      pltpu.sync_copy(dst_hbm.at[w, pl.ds(base, CB)], didx_v)
      pltpu.sync_copy(norm_hbm.at[w, pl.ds(base, CB)], norm_v)

      def body(i, _):
        pltpu.async_copy(xw_hbm.at[sidx_v.at[i]], rows_v, sem).wait()

        def scale(rb, _):
          nv16 = norm_v[i, pl.ds(rb * 16, 16)]
          for rr in range(16):
            r = rb * 16 + rr
            nv = nv16[rr]
            for j in range(D // 16):
              rows_v[r, pl.ds(j * 16, 16)] = rows_v[r, pl.ds(j * 16, 16)] * nv
          return 0

        lax.fori_loop(0, K // 16, scale, 0)
        pltpu.sync_copy(rows_v, acc_sh.at[didx_v.at[i]], add=True)
        return 0

      lax.fori_loop(0, CB, body, 0)
      return 0

    plsc.subcore_barrier()
    lax.fori_loop(0, C // CB, group, 0)
    plsc.subcore_barrier()
    pltpu.sync_copy(acc_sh.at[pl.ds(s * STRIPE, STRIPE)],
                    out_hbm.at[c, pl.ds(s * STRIPE, STRIPE)])

  return k(xw, src3, dst3, norm3)


_RB = 400
_NB = N_NODES // _RB


def _tc_matmul(x, W, dv):
  def body(x_ref, w_ref, dv_ref, o_ref):
    o_ref[...] = jnp.dot(x_ref[...], w_ref[...],
                         preferred_element_type=jnp.float32) * dv_ref[...]

  return pl.pallas_call(
      body,
      grid=(_NB,),
      in_specs=[
          pl.BlockSpec((_RB, D), lambda i: (i, 0)),
          pl.BlockSpec((D, D), lambda i: (0, 0)),
          pl.BlockSpec((_RB, 1), lambda i: (i, 0)),
      ],
      out_specs=pl.BlockSpec((_RB, D), lambda i: (i, 0)),
      out_shape=jax.ShapeDtypeStruct((N_NODES, D), jnp.float32),
  )(x, W, dv)


def _tc_mid(parts, y, dv, b, W2):
  def body(p0_ref, p1_ref, y_ref, dv_ref, b_ref, w2_ref, o_ref):
    h = (p0_ref[0] + p1_ref[0] + y_ref[...]) * dv_ref[...] + b_ref[...]
    h = jnp.maximum(h, 0.0)
    o_ref[...] = jnp.dot(h, w2_ref[...],
                         preferred_element_type=jnp.float32) * dv_ref[...]

  return pl.pallas_call(
      body,
      grid=(_NB,),
      in_specs=[
          pl.BlockSpec((1, _RB, D), lambda i: (0, i, 0)),
          pl.BlockSpec((1, _RB, D), lambda i: (1, i, 0)),
          pl.BlockSpec((_RB, D), lambda i: (i, 0)),
          pl.BlockSpec((_RB, 1), lambda i: (i, 0)),
          pl.BlockSpec((1, D), lambda i: (0, 0)),
          pl.BlockSpec((D, D), lambda i: (0, 0)),
      ],
      out_specs=pl.BlockSpec((_RB, D), lambda i: (i, 0)),
      out_shape=jax.ShapeDtypeStruct((N_NODES, D), jnp.float32),
  )(parts, parts, y, dv, b, W2)


def _tc_final(parts, y, dv, b, bid, speed, Ws, bs, Wl1g, Wl1v, bl1, Wl2, bl2):
  def body(p0_ref, p1_ref, y_ref, dv_ref, b_ref, bid_ref, speed_ref, ws_ref,
           bs_ref, wl1g_ref, wl1v_ref, bl1_ref, wl2_ref, bl2_ref, o_ref,
           acc_ref):
    i = pl.program_id(0)

    @pl.when(i == 0)
    def _():
      acc_ref[...] = jnp.full((N_GRAPHS, D), -jnp.inf, jnp.float32)

    h = (p0_ref[0] + p1_ref[0] + y_ref[...]) * dv_ref[...] + b_ref[...]
    h = jnp.maximum(h, 0.0)
    bid = bid_ref[...]
    for g in range(N_GRAPHS):
      mg = jnp.max(jnp.where(bid == g, h, -jnp.inf), axis=0, keepdims=True)
      acc_ref[pl.ds(g, 1), :] = jnp.maximum(acc_ref[pl.ds(g, 1), :], mg)

    @pl.when(i == _NB - 1)
    def _():
      gmax = acc_ref[...]
      v = speed_ref[...] * ws_ref[...] + bs_ref[...]
      hh = jnp.dot(gmax, wl1g_ref[...], preferred_element_type=jnp.float32)
      hh = hh + jnp.dot(v, wl1v_ref[...], preferred_element_type=jnp.float32)
      hh = jnp.maximum(hh + bl1_ref[...], 0.0)
      o_ref[...] = jnp.dot(hh, wl2_ref[...],
                           preferred_element_type=jnp.float32) + bl2_ref[...]

  return pl.pallas_call(
      body,
      grid=(_NB,),
      in_specs=[
          pl.BlockSpec((1, _RB, D), lambda i: (0, i, 0)),
          pl.BlockSpec((1, _RB, D), lambda i: (1, i, 0)),
          pl.BlockSpec((_RB, D), lambda i: (i, 0)),
          pl.BlockSpec((_RB, 1), lambda i: (i, 0)),
          pl.BlockSpec((1, D), lambda i: (0, 0)),
          pl.BlockSpec((_RB, 1), lambda i: (i, 0)),
          pl.BlockSpec((N_GRAPHS, 1), lambda i: (0, 0)),
          pl.BlockSpec((1, 4), lambda i: (0, 0)),
          pl.BlockSpec((1, 4), lambda i: (0, 0)),
          pl.BlockSpec((D, N_GRAPHS), lambda i: (0, 0)),
          pl.BlockSpec((4, N_GRAPHS), lambda i: (0, 0)),
          pl.BlockSpec((1, N_GRAPHS), lambda i: (0, 0)),
          pl.BlockSpec((N_GRAPHS, N_GRAPHS), lambda i: (0, 0)),
          pl.BlockSpec((1, N_GRAPHS), lambda i: (0, 0)),
      ],
      out_specs=pl.BlockSpec((N_GRAPHS, N_GRAPHS), lambda i: (0, 0)),
      out_shape=jax.ShapeDtypeStruct((N_GRAPHS, N_GRAPHS), jnp.float32),
      scratch_shapes=[pltpu.VMEM((N_GRAPHS, D), jnp.float32)],
  )(parts, parts, y, dv, b, bid, speed, Ws, bs, Wl1g, Wl1v, bl1, Wl2, bl2)


def kernel(x, edge_index, edge_weight, batch_ids, speed,
           W1, b1, W2, b2, Ws, bs, Wl1, bl1, Wl2, bl2):
  src = edge_index[0].astype(jnp.int32)
  dst = edge_index[1].astype(jnp.int32)
  ew = edge_weight.astype(jnp.float32)
  pad = E_PAD - N_EDGES
  src3 = jnp.pad(src, (0, pad)).reshape(NW, C, K)
  dst3 = jnp.pad(dst, (0, pad)).reshape(NW, C, K)
  w3 = jnp.pad(ew, (0, pad)).reshape(NW, C, K)
  bid = batch_ids.astype(jnp.int32).reshape(N_NODES, 1)

  deg_parts = _sc_deg(dst3, w3)
  deg = deg_parts[0, :N_NODES] + deg_parts[1, :N_NODES] + 1.0
  dinv = jnp.where(deg > 0, lax.rsqrt(deg), 0.0)
  dv = dinv.reshape(N_NODES, 1)

  y1 = _tc_matmul(x, W1, dv)
  p1 = _sc_spmm(y1, src3, dst3, w3)
  y2 = _tc_mid(p1, y1, dv, b1.reshape(1, D), W2)
  p2 = _sc_spmm(y2, src3, dst3, w3)

  out = _tc_final(p2, y2, dv, b2.reshape(1, D), bid, speed,
                  Ws, bs.reshape(1, 4), Wl1[:D], Wl1[D:], bl1.reshape(1, N_GRAPHS),
                  Wl2, bl2.reshape(1, N_GRAPHS))
  return out


# speedup vs baseline: 8.5163x; 1.1438x over previous
import functools

import jax
import jax.numpy as jnp
from jax import lax
from jax.experimental import pallas as pl
from jax.experimental.pallas import tpu as pltpu
from jax.experimental.pallas import tpu_sc as plsc

N_NODES = 10000
N_EDGES = 320000
N_GRAPHS = 16
D = 128

NC = 2
NS = 16
NW = NC * NS
K = 128
C = 80
CB = 16
E_PAD = NW * C * K
N_PAD = 10240
STRIPE = N_PAD // NS

_mesh = lambda: plsc.VectorSubcoreMesh(
    core_axis_name="c", subcore_axis_name="s", num_cores=NC, num_subcores=NS)


def _sc_deg(dst3, w3):
  @functools.partial(
      pl.kernel,
      out_type=jax.ShapeDtypeStruct((NC, N_PAD), jnp.float32),
      mesh=_mesh(),
      scratch_types=[
          pltpu.VMEM((C, K), jnp.int32),
          pltpu.VMEM((C, K), jnp.float32),
          pltpu.VMEM((STRIPE,), jnp.float32),
          pltpu.VMEM_SHARED((N_PAD,), jnp.float32),
          pltpu.SemaphoreType.DMA,
      ],
  )
  def k(dst_hbm, w_hbm, out_hbm, idx_v, w_v, zero_v, acc_sh, sem):
    c = lax.axis_index("c")
    s = lax.axis_index("s")
    w = s * NC + c
    z16 = jnp.zeros((16,), jnp.float32)

    def zinit(i, _):
      zero_v[pl.ds(i * 16, 16)] = z16
      return 0

    lax.fori_loop(0, STRIPE // 16, zinit, 0)
    pltpu.sync_copy(zero_v, acc_sh.at[pl.ds(s * STRIPE, STRIPE)])
    plsc.subcore_barrier()

    pltpu.sync_copy(dst_hbm.at[w], idx_v)
    pltpu.sync_copy(w_hbm.at[w], w_v)

    def body(bi, _):
      for j in range(8):
        i = bi * 8 + j
        pltpu.async_copy(w_v.at[i], acc_sh.at[idx_v.at[i]], sem, add=True)
      for j in range(8):
        pltpu.make_async_copy(w_v.at[0], acc_sh.at[idx_v.at[0]], sem).wait()
      return 0

    lax.fori_loop(0, C // 8, body, 0)
    plsc.subcore_barrier()
    pltpu.sync_copy(acc_sh.at[pl.ds(s * STRIPE, STRIPE)],
                    out_hbm.at[c, pl.ds(s * STRIPE, STRIPE)])

  return k(dst3, w3)


def _sc_spmm(xw, src3, dst3, norm3):
  @functools.partial(
      pl.kernel,
      out_type=jax.ShapeDtypeStruct((NC, N_PAD, D), jnp.float32),
      mesh=_mesh(),
      scratch_types=[
          pltpu.VMEM((CB, K), jnp.int32),
          pltpu.VMEM((CB, K), jnp.int32),
          pltpu.VMEM((CB, K), jnp.float32),
          pltpu.VMEM((K, D), jnp.float32),
          pltpu.VMEM((K, D), jnp.float32),
          pltpu.VMEM_SHARED((N_PAD, D), jnp.float32),
          pltpu.SemaphoreType.DMA,
          pltpu.SemaphoreType.DMA,
          pltpu.SemaphoreType.DMA,
          pltpu.SemaphoreType.DMA,
      ],
  )
  def k(xw_hbm, src_hbm, dst_hbm, norm_hbm, out_hbm,
        sidx_v, didx_v, norm_v, rows0, rows1, acc_sh,
        semg0, semg1, sems0, sems1):
    c = lax.axis_index("c")
    s = lax.axis_index("s")
    w = s * NC + c
    z16 = jnp.zeros((16,), jnp.float32)

    def zinit(i, _):
      for j in range(D // 16):
        rows0[i, pl.ds(j * 16, 16)] = z16
      return 0

    lax.fori_loop(0, K, zinit, 0)
    for t in range(STRIPE // K):
      pltpu.sync_copy(rows0, acc_sh.at[pl.ds(s * STRIPE + t * K, K)])

    def scale(buf, li):
      def srb(rb, _):
        nv16 = norm_v[li, pl.ds(rb * 16, 16)]
        for rr in range(16):
          r = rb * 16 + rr
          nv = nv16[rr]
          for j in range(D // 16):
            buf[r, pl.ds(j * 16, 16)] = buf[r, pl.ds(j * 16, 16)] * nv
        return 0

      lax.fori_loop(0, K // 16, srb, 0)

    def group(g, _):
      base = g * CB
      pltpu.sync_copy(src_hbm.at[w, pl.ds(base, CB)], sidx_v)
      pltpu.sync_copy(dst_hbm.at[w, pl.ds(base, CB)], didx_v)
      pltpu.sync_copy(norm_hbm.at[w, pl.ds(base, CB)], norm_v)
      pltpu.async_copy(xw_hbm.at[sidx_v.at[0]], rows0, semg0)

      def pair(t, _):
        li = 2 * t
        pltpu.make_async_copy(xw_hbm.at[sidx_v.at[li]], rows0, semg0).wait()

        @pl.when(t > 0)
        def _():
          pltpu.make_async_copy(rows1, acc_sh.at[didx_v.at[li]], sems1).wait()

        g1 = pltpu.async_copy(xw_hbm.at[sidx_v.at[li + 1]], rows1, semg1)
        scale(rows0, li)
        sc0 = pltpu.async_copy(rows0, acc_sh.at[didx_v.at[li]], sems0,
                               add=True)
        g1.wait()
        sc0.wait()

        @pl.when(t < CB // 2 - 1)
        def _():
          pltpu.async_copy(xw_hbm.at[sidx_v.at[li + 2]], rows0, semg0)

        scale(rows1, li + 1)
        pltpu.async_copy(rows1, acc_sh.at[didx_v.at[li + 1]], sems1, add=True)
        return 0

      lax.fori_loop(0, CB // 2, pair, 0)
      pltpu.make_async_copy(rows1, acc_sh.at[didx_v.at[0]], sems1).wait()
      return 0

    plsc.subcore_barrier()
    lax.fori_loop(0, C // CB, group, 0)
    plsc.subcore_barrier()
    pltpu.sync_copy(acc_sh.at[pl.ds(s * STRIPE, STRIPE)],
                    out_hbm.at[c, pl.ds(s * STRIPE, STRIPE)])

  return k(xw, src3, dst3, norm3)


_RB = 400
_NB = N_NODES // _RB


def _tc_matmul(x, W, dv):
  def body(x_ref, w_ref, dv_ref, o_ref):
    o_ref[...] = jnp.dot(x_ref[...], w_ref[...],
                         preferred_element_type=jnp.float32) * dv_ref[...]

  return pl.pallas_call(
      body,
      grid=(_NB,),
      in_specs=[
          pl.BlockSpec((_RB, D), lambda i: (i, 0)),
          pl.BlockSpec((D, D), lambda i: (0, 0)),
          pl.BlockSpec((_RB, 1), lambda i: (i, 0)),
      ],
      out_specs=pl.BlockSpec((_RB, D), lambda i: (i, 0)),
      out_shape=jax.ShapeDtypeStruct((N_NODES, D), jnp.float32),
  )(x, W, dv)


def _tc_mid(parts, y, dv, b, W2):
  def body(p0_ref, p1_ref, y_ref, dv_ref, b_ref, w2_ref, o_ref):
    h = (p0_ref[0] + p1_ref[0] + y_ref[...]) * dv_ref[...] + b_ref[...]
    h = jnp.maximum(h, 0.0)
    o_ref[...] = jnp.dot(h, w2_ref[...],
                         preferred_element_type=jnp.float32) * dv_ref[...]

  return pl.pallas_call(
      body,
      grid=(_NB,),
      in_specs=[
          pl.BlockSpec((1, _RB, D), lambda i: (0, i, 0)),
          pl.BlockSpec((1, _RB, D), lambda i: (1, i, 0)),
          pl.BlockSpec((_RB, D), lambda i: (i, 0)),
          pl.BlockSpec((_RB, 1), lambda i: (i, 0)),
          pl.BlockSpec((1, D), lambda i: (0, 0)),
          pl.BlockSpec((D, D), lambda i: (0, 0)),
      ],
      out_specs=pl.BlockSpec((_RB, D), lambda i: (i, 0)),
      out_shape=jax.ShapeDtypeStruct((N_NODES, D), jnp.float32),
  )(parts, parts, y, dv, b, W2)


def _tc_final(parts, y, dv, b, bid, speed, Ws, bs, Wl1g, Wl1v, bl1, Wl2, bl2):
  def body(p0_ref, p1_ref, y_ref, dv_ref, b_ref, bid_ref, speed_ref, ws_ref,
           bs_ref, wl1g_ref, wl1v_ref, bl1_ref, wl2_ref, bl2_ref, o_ref,
           acc_ref):
    i = pl.program_id(0)

    @pl.when(i == 0)
    def _():
      acc_ref[...] = jnp.full((N_GRAPHS, D), -jnp.inf, jnp.float32)

    h = (p0_ref[0] + p1_ref[0] + y_ref[...]) * dv_ref[...] + b_ref[...]
    h = jnp.maximum(h, 0.0)
    bid = bid_ref[...]
    for g in range(N_GRAPHS):
      mg = jnp.max(jnp.where(bid == g, h, -jnp.inf), axis=0, keepdims=True)
      acc_ref[pl.ds(g, 1), :] = jnp.maximum(acc_ref[pl.ds(g, 1), :], mg)

    @pl.when(i == _NB - 1)
    def _():
      gmax = acc_ref[...]
      v = speed_ref[...] * ws_ref[...] + bs_ref[...]
      hh = jnp.dot(gmax, wl1g_ref[...], preferred_element_type=jnp.float32)
      hh = hh + jnp.dot(v, wl1v_ref[...], preferred_element_type=jnp.float32)
      hh = jnp.maximum(hh + bl1_ref[...], 0.0)
      o_ref[...] = jnp.dot(hh, wl2_ref[...],
                           preferred_element_type=jnp.float32) + bl2_ref[...]

  return pl.pallas_call(
      body,
      grid=(_NB,),
      in_specs=[
          pl.BlockSpec((1, _RB, D), lambda i: (0, i, 0)),
          pl.BlockSpec((1, _RB, D), lambda i: (1, i, 0)),
          pl.BlockSpec((_RB, D), lambda i: (i, 0)),
          pl.BlockSpec((_RB, 1), lambda i: (i, 0)),
          pl.BlockSpec((1, D), lambda i: (0, 0)),
          pl.BlockSpec((_RB, 1), lambda i: (i, 0)),
          pl.BlockSpec((N_GRAPHS, 1), lambda i: (0, 0)),
          pl.BlockSpec((1, 4), lambda i: (0, 0)),
          pl.BlockSpec((1, 4), lambda i: (0, 0)),
          pl.BlockSpec((D, N_GRAPHS), lambda i: (0, 0)),
          pl.BlockSpec((4, N_GRAPHS), lambda i: (0, 0)),
          pl.BlockSpec((1, N_GRAPHS), lambda i: (0, 0)),
          pl.BlockSpec((N_GRAPHS, N_GRAPHS), lambda i: (0, 0)),
          pl.BlockSpec((1, N_GRAPHS), lambda i: (0, 0)),
      ],
      out_specs=pl.BlockSpec((N_GRAPHS, N_GRAPHS), lambda i: (0, 0)),
      out_shape=jax.ShapeDtypeStruct((N_GRAPHS, N_GRAPHS), jnp.float32),
      scratch_shapes=[pltpu.VMEM((N_GRAPHS, D), jnp.float32)],
  )(parts, parts, y, dv, b, bid, speed, Ws, bs, Wl1g, Wl1v, bl1, Wl2, bl2)


def kernel(x, edge_index, edge_weight, batch_ids, speed,
           W1, b1, W2, b2, Ws, bs, Wl1, bl1, Wl2, bl2):
  src = edge_index[0].astype(jnp.int32)
  dst = edge_index[1].astype(jnp.int32)
  ew = edge_weight.astype(jnp.float32)
  pad = E_PAD - N_EDGES
  src3 = jnp.pad(src, (0, pad)).reshape(NW, C, K)
  dst3 = jnp.pad(dst, (0, pad)).reshape(NW, C, K)
  w3 = jnp.pad(ew, (0, pad)).reshape(NW, C, K)
  bid = batch_ids.astype(jnp.int32).reshape(N_NODES, 1)

  deg_parts = _sc_deg(dst3, w3)
  deg = deg_parts[0, :N_NODES] + deg_parts[1, :N_NODES] + 1.0
  dinv = jnp.where(deg > 0, lax.rsqrt(deg), 0.0)
  dv = dinv.reshape(N_NODES, 1)

  y1 = _tc_matmul(x, W1, dv)
  p1 = _sc_spmm(y1, src3, dst3, w3)
  y2 = _tc_mid(p1, y1, dv, b1.reshape(1, D), W2)
  p2 = _sc_spmm(y2, src3, dst3, w3)

  out = _tc_final(p2, y2, dv, b2.reshape(1, D), bid, speed,
                  Ws, bs.reshape(1, 4), Wl1[:D], Wl1[D:], bl1.reshape(1, N_GRAPHS),
                  Wl2, bl2.reshape(1, N_GRAPHS))
  return out
